# baseline (speedup 1.0000x reference)
import functools

import jax
import jax.numpy as jnp
from jax import lax
from jax.experimental import pallas as pl
from jax.experimental.pallas import tpu as pltpu
from jax.experimental.pallas import tpu_sc as plsc

VOCAB = 1000000
EMBED = 64
BATCH = 16384

_info = plsc.get_sparse_core_info()
_NC = _info.num_cores
_NS = _info.num_subcores
_NW = _NC * _NS

_mesh = plsc.VectorSubcoreMesh(core_axis_name="c", subcore_axis_name="s")

_NBLK = 124
_BW = 256


@functools.partial(
    pl.kernel,
    mesh=_mesh,
    out_type=jax.ShapeDtypeStruct((VOCAB * EMBED,), jnp.float32),
    scratch_types=[
        pltpu.VMEM((EMBED, _BW), jnp.float32),
        pltpu.VMEM((EMBED, _BW), jnp.float32),
        pltpu.VMEM((_BW * EMBED,), jnp.float32),
        pltpu.VMEM((_BW * EMBED,), jnp.float32),
        pltpu.SemaphoreType.DMA,
        pltpu.SemaphoreType.DMA,
        pltpu.SemaphoreType.DMA,
        pltpu.SemaphoreType.DMA,
    ],
    compiler_params=pltpu.CompilerParams(needs_layout_passes=False),
)
def _relayout(table_t_hbm, tail_hbm, dense_hbm, buf0, buf1, stg0, stg1,
              sem_i0, sem_i1, sem_o0, sem_o1):
    wid = lax.axis_index("s") * _NC + lax.axis_index("c")
    iota16 = lax.iota(jnp.int32, 16)
    iota64 = iota16 * EMBED
    rot = [(iota16 + d) & 15 for d in range(16)]
    pos0 = [iota64 + rot[d] for d in range(16)]
    bufs = (buf0, buf1)
    stgs = (stg0, stg1)
    sems_i = (sem_i0, sem_i1)
    sems_o = (sem_o0, sem_o1)

    def pcol(b):
        p = jnp.minimum(wid + _NW * b, (VOCAB - EMBED) // _BW - 1) * _BW
        return pl.multiple_of(p, _BW)

    def issue_in(b, x):
        pltpu.async_copy(table_t_hbm.at[:, pl.ds(pcol(b), _BW)], bufs[x], sems_i[x])

    issue_in(0, 0)
    issue_in(1, 1)

    @pl.loop(0, _NBLK, step=2)
    def blk_loop(i):
        for x in (0, 1):
            b = i + x
            @pl.when(i >= 2)
            def _():
                pltpu.make_async_copy(
                    stgs[x], dense_hbm.at[pl.ds(0, _BW * EMBED)], sems_o[x]
                ).wait()

            pltpu.make_async_copy(
                table_t_hbm.at[:, pl.ds(0, _BW)], bufs[x], sems_i[x]
            ).wait()

            @plsc.parallel_loop(0, _BW // 16, unroll=4)
            def l2b_loop(l2b):
                lvec = iota16 + l2b * 16
                lbase = l2b * (16 * EMBED)
                for cg in range(EMBED // 16):
                    bufv = bufs[x].at[pl.ds(cg * 16, 16), :]
                    cgbase = lbase + cg * 16
                    for d in range(16):
                        val = plsc.load_gather(bufv, [rot[d], lvec])
                        plsc.store_scatter(stgs[x], [pos0[d] + cgbase], val)

            pltpu.async_copy(
                stgs[x], dense_hbm.at[pl.ds(pcol(b) * EMBED, _BW * EMBED)], sems_o[x]
            )

            @pl.when(i < _NBLK - 2)
            def _():
                issue_in(b + 2, x)

    for x in (0, 1):
        pltpu.make_async_copy(
            stgs[x], dense_hbm.at[pl.ds(0, _BW * EMBED)], sems_o[x]
        ).wait()

    @pl.when(wid == 0)
    def _():
        pltpu.sync_copy(tail_hbm, stg0.at[pl.ds(0, EMBED * EMBED)])
        pltpu.sync_copy(
            stg0.at[pl.ds(0, EMBED * EMBED)],
            dense_hbm.at[pl.ds((VOCAB - EMBED) * EMBED, EMBED * EMBED)],
        )


_RPW = BATCH // _NW
_CH = 128
_NCHUNK = _RPW // _CH


@functools.partial(
    pl.kernel,
    mesh=_mesh,
    out_type=(
        jax.ShapeDtypeStruct((EMBED * BATCH,), jnp.float32),
        jax.ShapeDtypeStruct((EMBED * BATCH,), jnp.float32),
    ),
    scratch_types=[
        pltpu.VMEM((_CH,), jnp.int32),
        pltpu.VMEM((_CH,), jnp.int32),
        pltpu.VMEM((_CH, 128), jnp.float32),
        pltpu.VMEM((EMBED * _CH,), jnp.float32),
        pltpu.SemaphoreType.DMA,
    ],
    compiler_params=pltpu.CompilerParams(needs_layout_passes=False),
)
def _lookup(center_hbm, context_hbm, table2_hbm, out_c_hbm, out_x_hbm,
            idx_rows, idx_m, dst, stg, sem):
    wid = lax.axis_index("s") * _NC + lax.axis_index("c")
    iota16 = lax.iota(jnp.int32, 16)

    for in_ref, out_ref in ((center_hbm, out_c_hbm), (context_hbm, out_x_hbm)):

        @pl.loop(0, _NCHUNK)
        def chunk_loop(ch):
            s_prime = wid * _NCHUNK + ch
            base_row = s_prime * _CH
            pltpu.sync_copy(in_ref.at[pl.ds(base_row, _CH)], idx_rows)
            for rb in range(_CH // 16):
                v = idx_rows[pl.ds(rb * 16, 16)]
                idx_m[pl.ds(rb * 16, 16)] = v >> 1
            pltpu.async_copy(table2_hbm.at[idx_m], dst, sem).wait()
            for rb in range(_CH // 16):
                v = idx_rows[pl.ds(rb * 16, 16)]
                rows = iota16 + (rb * 16)
                half = (v & 1) << 6
                for c in range(EMBED):
                    stg[pl.ds(c * _CH + rb * 16, 16)] = plsc.load_gather(
                        dst, [rows, half + c]
                    )
            for k in range(EMBED // 8):
                pltpu.sync_copy(
                    stg.at[pl.ds(k * 1024, 1024)],
                    out_ref.at[pl.ds(k * 131072 + s_prime * 1024, 1024)],
                )


def kernel(center, context, word_embeds):
    dense_flat = _relayout(word_embeds.T, word_embeds[VOCAB - EMBED:].reshape(-1))
    table2 = dense_flat.reshape(VOCAB // 2, 128)
    buf_c, buf_x = _lookup(center, context, table2)

    def fix(buf):
        return buf.reshape(8, 128, 8, 128).transpose(1, 3, 0, 2).reshape(BATCH, EMBED)

    return (fix(buf_c), fix(buf_x))


# speedup vs baseline: 1.5770x; 1.5770x over previous
import functools

import jax
import jax.numpy as jnp
from jax import lax
from jax.experimental import pallas as pl
from jax.experimental.pallas import tpu as pltpu
from jax.experimental.pallas import tpu_sc as plsc

VOCAB = 1000000
EMBED = 64
BATCH = 16384

_info = plsc.get_sparse_core_info()
_NC = _info.num_cores
_NS = _info.num_subcores
_NW = _NC * _NS

_mesh = plsc.VectorSubcoreMesh(core_axis_name="c", subcore_axis_name="s")

_NBLK = 124
_BW = 256


@functools.partial(
    pl.kernel,
    mesh=_mesh,
    out_type=jax.ShapeDtypeStruct((VOCAB * EMBED,), jnp.float32),
    scratch_types=[
        pltpu.VMEM((EMBED, _BW), jnp.float32),
        pltpu.VMEM((EMBED, _BW), jnp.float32),
        pltpu.VMEM((_BW * EMBED,), jnp.float32),
        pltpu.VMEM((_BW * EMBED,), jnp.float32),
        pltpu.SemaphoreType.DMA,
        pltpu.SemaphoreType.DMA,
        pltpu.SemaphoreType.DMA,
        pltpu.SemaphoreType.DMA,
    ],
    compiler_params=pltpu.CompilerParams(needs_layout_passes=False),
)
def _relayout(table_t_hbm, tail_hbm, dense_hbm, buf0, buf1, stg0, stg1,
              sem_i0, sem_i1, sem_o0, sem_o1):
    wid = lax.axis_index("s") * _NC + lax.axis_index("c")
    iota16 = lax.iota(jnp.int32, 16)
    iota64 = iota16 * EMBED
    rot = [(iota16 + d) & 15 for d in range(16)]
    pos0 = [iota64 + rot[d] for d in range(16)]
    bufs = (buf0, buf1)
    stgs = (stg0, stg1)
    sems_i = (sem_i0, sem_i1)
    sems_o = (sem_o0, sem_o1)

    def pcol(b):
        p = jnp.minimum(wid + _NW * b, (VOCAB - EMBED) // _BW - 1) * _BW
        return pl.multiple_of(p, _BW)

    def issue_in(b, x):
        pltpu.async_copy(table_t_hbm.at[:, pl.ds(pcol(b), _BW)], bufs[x], sems_i[x])

    issue_in(0, 0)
    issue_in(1, 1)

    @pl.loop(0, _NBLK, step=2)
    def blk_loop(i):
        for x in (0, 1):
            b = i + x
            @pl.when(i >= 2)
            def _():
                pltpu.make_async_copy(
                    stgs[x], dense_hbm.at[pl.ds(0, _BW * EMBED)], sems_o[x]
                ).wait()

            pltpu.make_async_copy(
                table_t_hbm.at[:, pl.ds(0, _BW)], bufs[x], sems_i[x]
            ).wait()

            @pl.loop(0, _BW // 16, unroll=4)
            def l2b_loop(l2b):
                lvec = iota16 + l2b * 16
                lbase = l2b * (16 * EMBED)
                for cg in range(EMBED // 16):
                    bufv = bufs[x].at[pl.ds(cg * 16, 16), :]
                    cgbase = lbase + cg * 16
                    for d in range(16):
                        val = plsc.load_gather(bufv, [rot[d], lvec])
                        plsc.store_scatter(stgs[x], [pos0[d] + cgbase], val)

            pltpu.async_copy(
                stgs[x], dense_hbm.at[pl.ds(pcol(b) * EMBED, _BW * EMBED)], sems_o[x]
            )

            @pl.when(i < _NBLK - 2)
            def _():
                issue_in(b + 2, x)

    for x in (0, 1):
        pltpu.make_async_copy(
            stgs[x], dense_hbm.at[pl.ds(0, _BW * EMBED)], sems_o[x]
        ).wait()

    @pl.when(wid == 0)
    def _():
        pltpu.sync_copy(tail_hbm, stg0.at[pl.ds(0, EMBED * EMBED)])
        pltpu.sync_copy(
            stg0.at[pl.ds(0, EMBED * EMBED)],
            dense_hbm.at[pl.ds((VOCAB - EMBED) * EMBED, EMBED * EMBED)],
        )


_RPW = BATCH // _NW
_CH = 128
_NCHUNK = _RPW // _CH


@functools.partial(
    pl.kernel,
    mesh=_mesh,
    out_type=(
        jax.ShapeDtypeStruct((EMBED * BATCH,), jnp.float32),
        jax.ShapeDtypeStruct((EMBED * BATCH,), jnp.float32),
    ),
    scratch_types=[
        pltpu.VMEM((_CH,), jnp.int32),
        pltpu.VMEM((_CH,), jnp.int32),
        pltpu.VMEM((_CH, 128), jnp.float32),
        pltpu.VMEM((EMBED * _CH,), jnp.float32),
        pltpu.SemaphoreType.DMA,
    ],
    compiler_params=pltpu.CompilerParams(needs_layout_passes=False),
)
def _lookup(center_hbm, context_hbm, table2_hbm, out_c_hbm, out_x_hbm,
            idx_rows, idx_m, dst, stg, sem):
    wid = lax.axis_index("s") * _NC + lax.axis_index("c")
    iota16 = lax.iota(jnp.int32, 16)

    for in_ref, out_ref in ((center_hbm, out_c_hbm), (context_hbm, out_x_hbm)):

        @pl.loop(0, _NCHUNK)
        def chunk_loop(ch):
            s_prime = wid * _NCHUNK + ch
            base_row = s_prime * _CH
            pltpu.sync_copy(in_ref.at[pl.ds(base_row, _CH)], idx_rows)
            for rb in range(_CH // 16):
                v = idx_rows[pl.ds(rb * 16, 16)]
                idx_m[pl.ds(rb * 16, 16)] = v >> 1
            pltpu.async_copy(table2_hbm.at[idx_m], dst, sem).wait()
            for rb in range(_CH // 16):
                v = idx_rows[pl.ds(rb * 16, 16)]
                rows = iota16 + (rb * 16)
                half = (v & 1) << 6
                for c in range(EMBED):
                    stg[pl.ds(c * _CH + rb * 16, 16)] = plsc.load_gather(
                        dst, [rows, half + c]
                    )
            for k in range(EMBED // 8):
                pltpu.sync_copy(
                    stg.at[pl.ds(k * 1024, 1024)],
                    out_ref.at[pl.ds(k * 131072 + s_prime * 1024, 1024)],
                )


def kernel(center, context, word_embeds):
    dense_flat = _relayout(word_embeds.T, word_embeds[VOCAB - EMBED:].reshape(-1))
    table2 = dense_flat.reshape(VOCAB // 2, 128)
    buf_c, buf_x = _lookup(center, context, table2)

    def fix(buf):
        return buf.reshape(8, 128, 8, 128).transpose(1, 3, 0, 2).reshape(BATCH, EMBED)

    return (fix(buf_c), fix(buf_x))
